# baseline (speedup 1.0000x reference)
import jax
import jax.numpy as jnp
from jax.experimental import pallas as pl
from jax.experimental.pallas import tpu as pltpu

N_ROWS = 8192
N_COLS = 2048
HALF_ROWS = N_ROWS // 2
CHUNK_ROWS = 256
NCH = HALF_ROWS // CHUNK_ROWS
NBUF = 4
D = 2


def _alloc_out():
    def body(o_ref):
        pass

    return pl.pallas_call(
        body,
        out_specs=pl.BlockSpec(memory_space=pl.ANY),
        out_shape=jax.ShapeDtypeStruct((N_ROWS, N_COLS), jnp.float32),
    )()


def _tc_multiplex(x0, x1, x2, x3, sel_arr):
    def body(sel_ref, xa0, xa1, xa2, xa3, xb0, xb1, xb2, xb3, alias_in,
             out_h, *scratch):
        bufs = scratch[: 2 * NBUF]
        rsem = scratch[2 * NBUF : 4 * NBUF]
        wsem = scratch[4 * NBUF : 6 * NBUF]
        s = sel_ref[0]
        srcs_a = (xa0, xa1, xa2, xa3)
        srcs_b = (xb0, xb1, xb2, xb3)

        def copy_from(src_a, src_b):
            halves = (
                (src_a, out_h, 0, bufs[:NBUF], rsem[:NBUF], wsem[:NBUF]),
                (src_b, alias_in, HALF_ROWS, bufs[NBUF:], rsem[NBUF:],
                 wsem[NBUF:]),
            )

            def rd(h, i, wait):
                src, _, base, hbufs, hrsem, _ = halves[h]
                cp = pltpu.make_async_copy(
                    src.at[pl.ds(base + i * CHUNK_ROWS, CHUNK_ROWS)],
                    hbufs[i % NBUF], hrsem[i % NBUF])
                cp.wait() if wait else cp.start()

            def wr(h, i, wait):
                _, dst, base, hbufs, _, hwsem = halves[h]
                cp = pltpu.make_async_copy(
                    hbufs[i % NBUF],
                    dst.at[pl.ds(base + i * CHUNK_ROWS, CHUNK_ROWS)],
                    hwsem[i % NBUF])
                cp.wait() if wait else cp.start()

            for i in range(NCH + D):
                for h in range(2):
                    if i < NCH:
                        if i >= NBUF:
                            wr(h, i - NBUF, True)
                        rd(h, i, False)
                    if i >= D:
                        rd(h, i - D, True)
                        wr(h, i - D, False)
            for j in range(NCH - NBUF, NCH):
                for h in range(2):
                    wr(h, j, True)

        for j in range(4):
            @pl.when(s == j)
            def _(j=j):
                copy_from(srcs_a[j], srcs_b[j])

    return pl.pallas_call(
        body,
        in_specs=[pl.BlockSpec(memory_space=pltpu.SMEM)]
        + [pl.BlockSpec(memory_space=pl.ANY)] * 9,
        out_specs=pl.BlockSpec(memory_space=pl.ANY),
        out_shape=jax.ShapeDtypeStruct((N_ROWS, N_COLS), jnp.float32),
        input_output_aliases={9: 0},
        scratch_shapes=(
            [pltpu.VMEM((CHUNK_ROWS, N_COLS), jnp.float32)
             for _ in range(2 * NBUF)]
            + [pltpu.SemaphoreType.DMA for _ in range(4 * NBUF)]
        ),
    )(sel_arr, x0, x1, x2, x3, x0, x1, x2, x3, _alloc_out())


def kernel(x0, x1, x2, x3, sel):
    sel_arr = jnp.asarray(sel, dtype=jnp.int32).reshape((1,))
    return _tc_multiplex(x0, x1, x2, x3, sel_arr)
